# baseline (speedup 1.0000x reference)
import dataclasses
import functools

import jax
import jax.numpy as jnp
from jax import lax
from jax.experimental import pallas as pl
from jax.experimental.pallas import tpu as pltpu
from jax.experimental.pallas import tpu_sc as plsc

ROWS, COLS = 128, 32768
TOPK = 2048
LANES = 16
NTILES = 32
ROWS_PER_TILE = ROWS // NTILES
NV = COLS // LANES

DB1, DB2, DB3 = 11, 10, 10
NB1, NB2, NB3 = 1 << DB1, 1 << DB2, 1 << DB3
HIST_WORDS = NB1 * LANES


def _scan_hist(hist_ref, nbins, kth):
    ngroups = nbins // LANES

    def phase_a(g, carry):
        cnt, gsel, cnt_before, found = carry
        gi = ngroups - 1 - g
        acc = jnp.zeros((LANES,), jnp.int32)
        for j in range(LANES):
            acc = acc + hist_ref[pl.ds((gi * LANES + j) * LANES, LANES)]
        s = jnp.sum(acc)
        cross = jnp.logical_and(jnp.logical_not(found), cnt + s >= kth)
        gsel = jnp.where(cross, gi, gsel)
        cnt_before = jnp.where(cross, cnt, cnt_before)
        found = jnp.logical_or(found, cross)
        cnt = jnp.where(found, cnt, cnt + s)
        return cnt, gsel, cnt_before, found

    zero = jnp.int32(0)
    _, gsel, cnt_g, found_g = lax.fori_loop(
        0, ngroups, phase_a, (zero, zero, zero, False))

    def phase_b(j, carry):
        cnt, bsel, cnt_before, found = carry
        bi = gsel * LANES + (LANES - 1 - j)
        s = jnp.sum(hist_ref[pl.ds(bi * LANES, LANES)])
        cross = jnp.logical_and(jnp.logical_not(found), cnt + s >= kth)
        bsel = jnp.where(cross, bi, bsel)
        cnt_before = jnp.where(cross, cnt, cnt_before)
        found = jnp.logical_or(found, cross)
        cnt = jnp.where(found, cnt, cnt + s)
        return cnt, bsel, cnt_before, found

    _, bsel, cnt_b, found_b = lax.fori_loop(
        0, LANES, phase_b, (cnt_g, zero, zero, False))

    k_rem = kth - cnt_b
    return bsel, k_rem, jnp.logical_and(found_g, found_b)


UNROLL = 8


def _process_row(row_ref, hist_ref, buf_ref):
    lanes = lax.iota(jnp.int32, LANES)
    ones = jnp.ones((LANES,), jnp.int32)
    zeros = jnp.zeros((LANES,), jnp.int32)

    def clear(nwords):
        @pl.loop(0, nwords // LANES, step=UNROLL)
        def _(i):
            for u in range(UNROLL):
                hist_ref[pl.ds((i + u) * LANES, LANES)] = zeros

    clear(HIST_WORDS)

    @pl.loop(0, NV, step=UNROLL)
    def _(i):
        for u in range(UNROLL):
            xv = row_ref[pl.ds((i + u) * LANES, LANES)]
            key = plsc.bitcast(jnp.maximum(xv, 0.0), jnp.int32)
            m = xv > 0.0
            slot = (((key >> 20) & (NB1 - 1)) << 4) | lanes
            plsc.addupdate_scatter(hist_ref, [slot], ones, mask=m)

    b1, k1, f1 = _scan_hist(hist_ref, NB1, jnp.int32(TOPK))

    low_mask = (1 << 20) - 1

    def compact_cond(carry):
        return carry[0] < NV // 4

    def compact_step(carry):
        i, off = carry
        for u in range(4):
            xv = row_ref[pl.ds((4 * i + u) * LANES, LANES)]
            key = plsc.bitcast(jnp.maximum(xv, 0.0), jnp.int32)
            m = jnp.logical_and(xv > 0.0, ((key >> 20) & (NB1 - 1)) == b1)
            mi = jnp.where(m, 1, 0)
            rank = plsc.cumsum(mi) - mi
            plsc.store_scatter(buf_ref, [off + rank], key & low_mask,
                               mask=m)
            off = off + plsc.all_reduce_population_count(m)
        return i + 1, off

    _, offv = lax.while_loop(compact_cond, compact_step,
                             (jnp.int32(0), jnp.zeros((LANES,), jnp.int32)))
    n_cand = jnp.max(offv)
    zeros16 = jnp.zeros((LANES,), jnp.int32)
    for u in range(4):
        plsc.store_scatter(buf_ref, [offv + (u * LANES + lanes)], zeros16)
    nvb4 = (n_cand + 4 * LANES - 1) >> 6

    def bit_step(bi, prefix):
        trial = prefix | (1 << (19 - bi))

        def cnt_step(j, acc):
            kv = buf_ref[pl.ds(j * LANES, LANES)]
            acc = acc + jnp.where(kv >= trial, 1, 0)
            return acc

        accv = lax.fori_loop(0, 4 * nvb4, cnt_step,
                             jnp.zeros((LANES,), jnp.int32))
        return jnp.where(jnp.sum(accv) >= k1, trial, prefix)

    low20 = lax.fori_loop(0, 20, bit_step, jnp.int32(0))

    thresh = (b1 << 20) | low20
    thresh = jnp.where(f1, thresh, 0)
    tvec = jnp.full((LANES,), thresh, jnp.int32)

    @pl.loop(0, NV, step=UNROLL)
    def _(i):
        for u in range(UNROLL):
            xv = row_ref[pl.ds((i + u) * LANES, LANES)]
            xr = jnp.maximum(xv, 0.0)
            key = plsc.bitcast(xr, jnp.int32)
            row_ref[pl.ds((i + u) * LANES, LANES)] = jnp.where(
                key >= tvec, xr, 0.0)


def kernel(x):
    mesh = plsc.VectorSubcoreMesh(core_axis_name="c", subcore_axis_name="s")
    cp = pltpu.CompilerParams()
    if "needs_layout_passes" in pltpu.CompilerParams.__dataclass_fields__:
        cp = dataclasses.replace(cp, needs_layout_passes=False)

    @functools.partial(
        pl.kernel,
        out_type=jax.ShapeDtypeStruct((ROWS, COLS), jnp.float32),
        mesh=mesh,
        compiler_params=cp,
        scratch_types=[
            pltpu.VMEM((COLS,), jnp.float32),
            pltpu.VMEM((HIST_WORDS,), jnp.int32),
            pltpu.VMEM((COLS + 4 * LANES,), jnp.int32),
        ],
    )
    def run(x_hbm, out_hbm, row_ref, hist_ref, buf_ref):
        wid = lax.axis_index("s") * 2 + lax.axis_index("c")

        @pl.loop(0, ROWS_PER_TILE)
        def _(r):
            row = wid * ROWS_PER_TILE + r
            pltpu.sync_copy(x_hbm.at[row], row_ref)
            _process_row(row_ref, hist_ref, buf_ref)
            pltpu.sync_copy(row_ref, out_hbm.at[row])

    return run(x)


# speedup vs baseline: 3.2548x; 3.2548x over previous
import dataclasses
import functools

import jax
import jax.numpy as jnp
from jax import lax
from jax.experimental import pallas as pl
from jax.experimental.pallas import tpu as pltpu
from jax.experimental.pallas import tpu_sc as plsc

ROWS, COLS = 128, 32768
TOPK = 2048
LANES = 16
NTILES = 32
ROWS_PER_TILE = ROWS // NTILES
NV = COLS // LANES

DB1 = 10
NB1 = 1 << DB1
LOWB = 31 - DB1
HIST_WORDS = NB1 * LANES
UNROLL = 8


def _scan_hist(hist_ref, nbins, kth):
    ngroups = nbins // LANES

    def phase_a(g, carry):
        cnt, gsel, cnt_before, found = carry
        gi = ngroups - 1 - g
        acc = jnp.zeros((LANES,), jnp.int32)
        for j in range(LANES):
            acc = acc + hist_ref[pl.ds((gi * LANES + j) * LANES, LANES)]
        s = jnp.sum(acc)
        cross = jnp.logical_and(jnp.logical_not(found), cnt + s >= kth)
        gsel = jnp.where(cross, gi, gsel)
        cnt_before = jnp.where(cross, cnt, cnt_before)
        found = jnp.logical_or(found, cross)
        cnt = jnp.where(found, cnt, cnt + s)
        return cnt, gsel, cnt_before, found

    zero = jnp.int32(0)
    _, gsel, cnt_g, found_g = lax.fori_loop(
        0, ngroups, phase_a, (zero, zero, zero, False))

    def phase_b(j, carry):
        cnt, bsel, cnt_before, found = carry
        bi = gsel * LANES + (LANES - 1 - j)
        s = jnp.sum(hist_ref[pl.ds(bi * LANES, LANES)])
        cross = jnp.logical_and(jnp.logical_not(found), cnt + s >= kth)
        bsel = jnp.where(cross, bi, bsel)
        cnt_before = jnp.where(cross, cnt, cnt_before)
        found = jnp.logical_or(found, cross)
        cnt = jnp.where(found, cnt, cnt + s)
        return cnt, bsel, cnt_before, found

    _, bsel, cnt_b, found_b = lax.fori_loop(
        0, LANES, phase_b, (cnt_g, zero, zero, False))

    k_rem = kth - cnt_b
    return bsel, k_rem, jnp.logical_and(found_g, found_b)


def _process_row(row_ref, hist_ref, buf_ref):
    lanes = lax.iota(jnp.int32, LANES)
    ones = jnp.ones((LANES,), jnp.int32)
    zeros = jnp.zeros((LANES,), jnp.int32)

    @plsc.parallel_loop(0, HIST_WORDS // LANES, unroll=UNROLL)
    def _(i):
        hist_ref[pl.ds(i * LANES, LANES)] = zeros

    @plsc.parallel_loop(0, NV, unroll=UNROLL)
    def _(i):
        xv = row_ref[pl.ds(i * LANES, LANES)]
        key = plsc.bitcast(jnp.maximum(xv, 0.0), jnp.int32)
        m = xv > 0.0
        slot = (((key >> LOWB) & (NB1 - 1)) << 4) | lanes
        plsc.addupdate_scatter(hist_ref, [slot], ones, mask=m)

    b1, k1, f1 = _scan_hist(hist_ref, NB1, jnp.int32(TOPK))

    low_mask = (1 << LOWB) - 1
    b1v = jnp.full((LANES,), b1, jnp.int32)

    @plsc.parallel_loop(0, NV, unroll=UNROLL,
                        carry=jnp.zeros((LANES,), jnp.int32))
    def cntv(i, cv):
        xv = row_ref[pl.ds(i * LANES, LANES)]
        key = plsc.bitcast(jnp.maximum(xv, 0.0), jnp.int32)
        m = jnp.logical_and(xv > 0.0, ((key >> LOWB) & (NB1 - 1)) == b1v)
        pos = (cv << 4) | lanes
        plsc.store_scatter(buf_ref, [pos], key & low_mask, mask=m)
        return cv + jnp.where(m, 1, 0)

    nvb = jnp.max(cntv)

    def bit_step(bi, prefix):
        trial = prefix | (1 << (LOWB - 1 - bi))
        trialv = jnp.full((LANES,), trial, jnp.int32)

        @plsc.parallel_loop(0, nvb, unroll=4,
                            carry=(cntv, jnp.zeros((LANES,), jnp.int32)))
        def res(j, c):
            rem, acc = c
            kv = buf_ref[pl.ds(j * LANES, LANES)]
            valid = jnp.logical_and(kv >= trialv, rem > 0)
            return rem - 1, acc + jnp.where(valid, 1, 0)

        return jnp.where(jnp.sum(res[1]) >= k1, trial, prefix)

    low_bits = lax.fori_loop(0, LOWB, bit_step, jnp.int32(0))

    thresh = (b1 << LOWB) | low_bits
    thresh = jnp.where(f1, thresh, 0)
    tvec = jnp.full((LANES,), thresh, jnp.int32)

    @plsc.parallel_loop(0, NV, unroll=UNROLL)
    def _(i):
        xv = row_ref[pl.ds(i * LANES, LANES)]
        xr = jnp.maximum(xv, 0.0)
        key = plsc.bitcast(xr, jnp.int32)
        row_ref[pl.ds(i * LANES, LANES)] = jnp.where(key >= tvec, xr, 0.0)


def kernel(x):
    mesh = plsc.VectorSubcoreMesh(core_axis_name="c", subcore_axis_name="s")
    cp = pltpu.CompilerParams()
    if "needs_layout_passes" in pltpu.CompilerParams.__dataclass_fields__:
        cp = dataclasses.replace(cp, needs_layout_passes=False)

    @functools.partial(
        pl.kernel,
        out_type=jax.ShapeDtypeStruct((ROWS, COLS), jnp.float32),
        mesh=mesh,
        compiler_params=cp,
        scratch_types=[
            pltpu.VMEM((COLS,), jnp.float32),
            pltpu.VMEM((COLS,), jnp.float32),
            pltpu.VMEM((HIST_WORDS,), jnp.int32),
            pltpu.VMEM((COLS,), jnp.int32),
            pltpu.SemaphoreType.DMA,
            pltpu.SemaphoreType.DMA,
            pltpu.SemaphoreType.DMA,
            pltpu.SemaphoreType.DMA,
        ],
    )
    def run(x_hbm, out_hbm, row_a, row_b, hist_ref, buf_ref,
            sem_in_a, sem_in_b, sem_out_a, sem_out_b):
        wid = lax.axis_index("s") * 2 + lax.axis_index("c")
        base = wid * ROWS_PER_TILE

        bufs = [(row_a, sem_in_a, sem_out_a), (row_b, sem_in_b, sem_out_b)]

        pltpu.make_async_copy(x_hbm.at[base], row_a, sem_in_a).start()

        for r in range(ROWS_PER_TILE):
            cur, sin, sout = bufs[r % 2]
            pltpu.make_async_copy(x_hbm.at[base + r], cur, sin).wait()
            if r + 1 < ROWS_PER_TILE:
                nxt, snin, snout = bufs[(r + 1) % 2]
                if r >= 1:
                    pltpu.make_async_copy(
                        nxt, out_hbm.at[base + r - 1], snout).wait()
                pltpu.make_async_copy(x_hbm.at[base + r + 1], nxt,
                                      snin).start()
            _process_row(cur, hist_ref, buf_ref)
            pltpu.make_async_copy(cur, out_hbm.at[base + r], sout).start()

        for r in (ROWS_PER_TILE - 2, ROWS_PER_TILE - 1):
            cur, _, sout = bufs[r % 2]
            pltpu.make_async_copy(cur, out_hbm.at[base + r], sout).wait()

    return run(x)
